# baseline (speedup 1.0000x reference)
import functools

import jax
import jax.numpy as jnp
from jax import lax
from jax.experimental import pallas as pl
from jax.experimental.pallas import tpu as pltpu
from jax.experimental.pallas import tpu_sc as plsc

N = 10000
E = 320000
H = 8
C = 128

E_TOT = E + N
K = 32
TILES = 16
CHUNKS = 648
PAIRS = CHUNKS // 2
EPT = K * CHUNKS
E_PAD = EPT * TILES
N_PAD = 10240
ROWS_PT = N_PAD // TILES
FIN_B = 64
ZB = 8
NBLK = 2000
HEADS_PER_SC = H // 2


def _pre_body(x_ref, wl_ref, wr_ref, xl_ref, xr_ref):
    x = x_ref[...]
    xl_ref[...] = jnp.dot(x, wl_ref[...], preferred_element_type=jnp.float32)
    xr_ref[...] = jnp.dot(x, wr_ref[...], preferred_element_type=jnp.float32)


def _pre(x, wl, wr):
    return pl.pallas_call(
        _pre_body,
        grid=(N // NBLK, H),
        in_specs=[
            pl.BlockSpec((NBLK, C), lambda i, h: (i, 0)),
            pl.BlockSpec((C, C), lambda i, h: (0, h)),
            pl.BlockSpec((C, C), lambda i, h: (0, h)),
        ],
        out_specs=[
            pl.BlockSpec((NBLK, C), lambda i, h: (h * (N // NBLK) + i, 0)),
            pl.BlockSpec((NBLK, C), lambda i, h: (h * (N // NBLK) + i, 0)),
        ],
        out_shape=[
            jax.ShapeDtypeStruct((H * N, C), jnp.float32),
            jax.ShapeDtypeStruct((H * N, C), jnp.float32),
        ],
    )(x, wl, wr)


def _mid_body(v_ref, b_ref, wl_ref, wr_ref, xl_ref, xr_ref):
    hmean = jnp.mean(v_ref[...], axis=0) + b_ref[...]
    h1 = jnp.where(hmean > 0, hmean, 0.01 * hmean)
    xl_ref[...] = jnp.dot(h1, wl_ref[...], preferred_element_type=jnp.float32)
    xr_ref[...] = jnp.dot(h1, wr_ref[...], preferred_element_type=jnp.float32)


def _mid(v, b, wl, wr):
    return pl.pallas_call(
        _mid_body,
        grid=(N // NBLK, H),
        in_specs=[
            pl.BlockSpec((H, NBLK, C), lambda i, h: (0, i, 0)),
            pl.BlockSpec((1, C), lambda i, h: (0, 0)),
            pl.BlockSpec((C, C), lambda i, h: (0, h)),
            pl.BlockSpec((C, C), lambda i, h: (0, h)),
        ],
        out_specs=[
            pl.BlockSpec((NBLK, C), lambda i, h: (h * (N // NBLK) + i, 0)),
            pl.BlockSpec((NBLK, C), lambda i, h: (h * (N // NBLK) + i, 0)),
        ],
        out_shape=[
            jax.ShapeDtypeStruct((H * N, C), jnp.float32),
            jax.ShapeDtypeStruct((H * N, C), jnp.float32),
        ],
    )(v, b, wl, wr)


def _post_body(v_ref, b_ref, o_ref):
    o_ref[...] = jnp.mean(v_ref[...], axis=0) + b_ref[...]


def _post(v, b):
    return pl.pallas_call(
        _post_body,
        grid=(N // NBLK,),
        in_specs=[
            pl.BlockSpec((H, NBLK, C), lambda i: (0, i, 0)),
            pl.BlockSpec((1, C), lambda i: (0, 0)),
        ],
        out_specs=pl.BlockSpec((NBLK, C), lambda i: (i, 0)),
        out_shape=jax.ShapeDtypeStruct((N, C), jnp.float32),
    )(v, b)


DROWS_PT = ROWS_PT // 8


def _sc_layer_body(xl_hbm, xr_hbm, src_hbm, dst_hbm, att_hbm, out_hbm,
                   att_v, sdsA, sddA, grxA, sdsB, sddB, grxB,
                   didxSA, didx8A, didxSB, didx8B,
                   xlA, xrA, xlB, xrB, dencA, dencB, denb, finb, zerob,
                   acc, den_acc, semIA, semIB, semLA, semRA, semLB, semRB,
                   semSA, semDA, semSB, semDB):
    c = lax.axis_index("c")
    s = lax.axis_index("s")
    edge_base = s * EPT
    node_base = s * ROWS_PT

    pltpu.sync_copy(att_hbm.at[pl.ds(c * (HEADS_PER_SC * C), HEADS_PER_SC * C)],
                    att_v)

    zero16 = jnp.zeros((16,), jnp.float32)
    lane = lax.iota(jnp.int32, 16)

    def _bcast(v, i):
        return jnp.take_along_axis(v, i, axis=0)

    def _hsum(v):
        for stp in (8, 4, 2, 1):
            v = v + jnp.take_along_axis(v, lane ^ stp, axis=0)
        return v

    def zb_row(r, _):
        for b in range(C // 16):
            zerob[r, pl.ds(b * 16, 16)] = zero16
        return 0
    lax.fori_loop(0, ZB, zb_row, 0)

    def zc_row(r, _):
        for b in range(C // 16):
            dencA[r, pl.ds(b * 16, 16)] = zero16
            dencB[r, pl.ds(b * 16, 16)] = zero16
        return 0
    lax.fori_loop(0, K, zc_row, 0)

    def zero_rows(ref, base, nrows):
        for j in range(nrows // ZB):
            pltpu.sync_copy(zerob, ref.at[pl.ds(base + j * ZB, ZB), :])

    zero_rows(acc, node_base, ROWS_PT)
    zero_rows(den_acc, s * DROWS_PT, DROWS_PT)
    plsc.subcore_barrier()

    def head_body(hl, _):
        h = c * HEADS_PER_SC + hl
        hoff = h * N
        ooff = h * N_PAD
        attv = [att_v[pl.ds(hl * C + b * 16, 16)] for b in range(C // 16)]

        def idx_issue(sds, sdd, semI, base):
            pltpu.async_copy(src_hbm.at[pl.ds(base, K)], sds, semI)
            pltpu.async_copy(dst_hbm.at[pl.ds(base, K)], sdd, semI)

        def idx_wait(sds, sdd, semI):
            pltpu.make_async_copy(src_hbm.at[pl.ds(0, K)], sds, semI).wait()
            pltpu.make_async_copy(dst_hbm.at[pl.ds(0, K)], sdd, semI).wait()

        def addoff(sds, sdd, grx):
            for g in range(K // 16):
                sl = pl.ds(g * 16, 16)
                sds[sl] = sds[sl] + hoff
                grx[sl] = sdd[sl] + hoff

        def gather_issue(sds, grx, xlb, xrb, semL, semR):
            pltpu.async_copy(xl_hbm.at[sds], xlb, semL)
            pltpu.async_copy(xr_hbm.at[grx], xrb, semR)

        def gather_wait(xlb, xrb, semL, semR):
            pltpu.make_async_copy(xl_hbm.at[pl.ds(0, K)], xlb, semL).wait()
            pltpu.make_async_copy(xr_hbm.at[pl.ds(0, K)], xrb, semR).wait()

        def scatter_wait(xlb, denc, semS, semD):
            pltpu.make_async_copy(xlb, acc.at[pl.ds(0, K)], semS).wait()
            pltpu.make_async_copy(denc, den_acc.at[pl.ds(0, K)], semD).wait()

        def compute_scatter(xlb, xrb, sdd, didxS, didx8, denc, semS, semD,
                            base, j):
            @pl.when(j > 0)
            def _():
                scatter_wait(xlb, denc, semS, semD)

            def group_body(g, _):
                d16 = sdd[pl.ds(g * 16, 16)]
                lanesel = d16 & 7
                didxS[pl.ds(g * 16, 16)] = d16
                didx8[pl.ds(g * 16, 16)] = d16 >> 3
                ids = lane + (base + g * 16)
                mf = jnp.where(ids < E_TOT, 1.0, 0.0)

                def edge_body(kk, _):
                    k = g * 16 + kk
                    kkv = jnp.full((16,), kk, jnp.int32)
                    xv = [xlb[k, pl.ds(b * 16, 16)] for b in range(C // 16)]
                    accv = zero16
                    for b in range(C // 16):
                        z = xv[b] + xrb[k, pl.ds(b * 16, 16)]
                        accv = accv + jnp.maximum(z, 0.2 * z) * attv[b]
                    ev = jnp.exp(_hsum(accv)) * _bcast(mf, kkv)
                    denc[k, pl.ds(0, 16)] = \
                        jnp.where(lane == _bcast(lanesel, kkv), ev, 0.0)
                    for b in range(C // 16):
                        xlb[k, pl.ds(b * 16, 16)] = ev * xv[b]
                    return 0
                lax.fori_loop(0, 16, edge_body, 0, unroll=4)
                return 0
            lax.fori_loop(0, K // 16, group_body, 0)
            pltpu.async_copy(xlb, acc.at[didxS], semS, add=True)
            pltpu.async_copy(denc, den_acc.at[didx8], semD, add=True)

        pltpu.sync_copy(src_hbm.at[pl.ds(edge_base, K)], sdsA)
        pltpu.sync_copy(dst_hbm.at[pl.ds(edge_base, K)], sddA)
        addoff(sdsA, sddA, grxA)
        gather_issue(sdsA, grxA, xlA, xrA, semLA, semRA)
        idx_issue(sdsB, sddB, semIB, edge_base + K)

        def pair_body(j, _):
            baseA = edge_base + (2 * j) * K
            more = j < PAIRS - 1
            idx_wait(sdsB, sddB, semIB)
            addoff(sdsB, sddB, grxB)
            gather_issue(sdsB, grxB, xlB, xrB, semLB, semRB)
            gather_wait(xlA, xrA, semLA, semRA)
            compute_scatter(xlA, xrA, sddA, didxSA, didx8A, dencA,
                            semSA, semDA, baseA, j)

            @pl.when(more)
            def _():
                idx_issue(sdsA, sddA, semIA, baseA + 2 * K)

            @pl.when(more)
            def _():
                idx_wait(sdsA, sddA, semIA)
                addoff(sdsA, sddA, grxA)
                gather_issue(sdsA, grxA, xlA, xrA, semLA, semRA)
            gather_wait(xlB, xrB, semLB, semRB)
            compute_scatter(xlB, xrB, sddB, didxSB, didx8B, dencB,
                            semSB, semDB, baseA + K, j)

            @pl.when(more)
            def _():
                idx_issue(sdsB, sddB, semIB, baseA + 3 * K)
            return 0
        lax.fori_loop(0, PAIRS, pair_body, 0)
        scatter_wait(xlA, dencA, semSA, semDA)
        scatter_wait(xlB, dencB, semSB, semDB)
        plsc.subcore_barrier()

        def fin_blk(j, _):
            rb = node_base + j * FIN_B
            pltpu.sync_copy(acc.at[pl.ds(rb, FIN_B), :], finb)
            pltpu.sync_copy(den_acc.at[pl.ds(rb // 8, FIN_B // 8), :], denb)

            def fin_g(g, _):
                ra = denb[2 * g, pl.ds(0, 16)]
                rbv = denb[2 * g + 1, pl.ds(0, 16)]
                shifted = jnp.take_along_axis(rbv, (lane + 8) & 15, axis=0)
                iv16 = 1.0 / jnp.where(lane < 8, ra, shifted)
                for kk in range(16):
                    r = g * 16 + kk
                    iv = _bcast(iv16, jnp.full((16,), kk, jnp.int32))
                    for b in range(C // 16):
                        finb[r, pl.ds(b * 16, 16)] = \
                            finb[r, pl.ds(b * 16, 16)] * iv
                return 0
            lax.fori_loop(0, FIN_B // 16, fin_g, 0)
            pltpu.sync_copy(finb, out_hbm.at[pl.ds(ooff + rb, FIN_B), :])
            zero_rows(acc, rb, FIN_B)
            return 0
        lax.fori_loop(0, ROWS_PT // FIN_B, fin_blk, 0)
        zero_rows(den_acc, s * DROWS_PT, DROWS_PT)
        plsc.subcore_barrier()
        return 0
    lax.fori_loop(0, HEADS_PER_SC, head_body, 0)


def _sc_layer(xl, xr, src, dst, att):
    mesh = plsc.VectorSubcoreMesh(core_axis_name="c", subcore_axis_name="s")
    return pl.kernel(
        _sc_layer_body,
        out_type=jax.ShapeDtypeStruct((H * N_PAD, C), jnp.float32),
        mesh=mesh,
        scratch_types=[
            pltpu.VMEM((HEADS_PER_SC * C,), jnp.float32),
            pltpu.VMEM((K,), jnp.int32),
            pltpu.VMEM((K,), jnp.int32),
            pltpu.VMEM((K,), jnp.int32),
            pltpu.VMEM((K,), jnp.int32),
            pltpu.VMEM((K,), jnp.int32),
            pltpu.VMEM((K,), jnp.int32),
            pltpu.VMEM((K,), jnp.int32),
            pltpu.VMEM((K,), jnp.int32),
            pltpu.VMEM((K,), jnp.int32),
            pltpu.VMEM((K,), jnp.int32),
            pltpu.VMEM((K, C), jnp.float32),
            pltpu.VMEM((K, C), jnp.float32),
            pltpu.VMEM((K, C), jnp.float32),
            pltpu.VMEM((K, C), jnp.float32),
            pltpu.VMEM((K, C), jnp.float32),
            pltpu.VMEM((K, C), jnp.float32),
            pltpu.VMEM((FIN_B // 8, C), jnp.float32),
            pltpu.VMEM((FIN_B, C), jnp.float32),
            pltpu.VMEM((ZB, C), jnp.float32),
            pltpu.VMEM_SHARED((N_PAD, C), jnp.float32),
            pltpu.VMEM_SHARED((N_PAD // 8, C), jnp.float32),
        ] + [pltpu.SemaphoreType.DMA] * 10,
    )(xl, xr, src, dst, att)


def kernel(x, edge_index, W_l1, W_r1, att1, b1, W_l2, W_r2, att2, b2):
    loop = jnp.arange(N, dtype=jnp.int32)
    src = jnp.concatenate([edge_index[0].astype(jnp.int32), loop])
    dst = jnp.concatenate([edge_index[1].astype(jnp.int32), loop])
    src = jnp.pad(src, (0, E_PAD - E_TOT))
    dst = jnp.pad(dst, (0, E_PAD - E_TOT))

    xl1, xr1 = _pre(x, W_l1, W_r1)
    v1 = _sc_layer(xl1, xr1, src, dst, att1.reshape(-1))
    v1 = v1.reshape(H, N_PAD, C)[:, :N, :]
    xl2, xr2 = _mid(v1, b1.reshape(1, C), W_l2, W_r2)
    v2 = _sc_layer(xl2, xr2, src, dst, att2.reshape(-1))
    v2 = v2.reshape(H, N_PAD, C)[:, :N, :]
    return _post(v2, b2.reshape(1, C))


# speedup vs baseline: 1.6060x; 1.6060x over previous
import functools

import jax
import jax.numpy as jnp
from jax import lax
from jax.experimental import pallas as pl
from jax.experimental.pallas import tpu as pltpu
from jax.experimental.pallas import tpu_sc as plsc

N = 10000
E = 320000
H = 8
C = 128

E_TOT = E + N
K = 32
TILES = 16
CHUNKS = 648
PAIRS = CHUNKS // 2
EPT = K * CHUNKS
E_PAD = EPT * TILES
N_PAD = 10240
ROWS_PT = N_PAD // TILES
FIN_B = 64
ZB = 8
NBLK = 2000
HEADS_PER_SC = H // 2


def _pre_body(x_ref, wl_ref, wr_ref, xl_ref, xr_ref):
    x = x_ref[...]
    xl_ref[...] = jnp.dot(x, wl_ref[...], preferred_element_type=jnp.float32)
    xr_ref[...] = jnp.dot(x, wr_ref[...], preferred_element_type=jnp.float32)


def _pre(x, wl, wr):
    return pl.pallas_call(
        _pre_body,
        grid=(N // NBLK, H),
        in_specs=[
            pl.BlockSpec((NBLK, C), lambda i, h: (i, 0)),
            pl.BlockSpec((C, C), lambda i, h: (0, h)),
            pl.BlockSpec((C, C), lambda i, h: (0, h)),
        ],
        out_specs=[
            pl.BlockSpec((NBLK, C), lambda i, h: (h * (N // NBLK) + i, 0)),
            pl.BlockSpec((NBLK, C), lambda i, h: (h * (N // NBLK) + i, 0)),
        ],
        out_shape=[
            jax.ShapeDtypeStruct((H * N, C), jnp.float32),
            jax.ShapeDtypeStruct((H * N, C), jnp.float32),
        ],
    )(x, wl, wr)


def _mid_body(v_ref, b_ref, wl_ref, wr_ref, xl_ref, xr_ref):
    hmean = jnp.mean(v_ref[...], axis=0) + b_ref[...]
    h1 = jnp.where(hmean > 0, hmean, 0.01 * hmean)
    xl_ref[...] = jnp.dot(h1, wl_ref[...], preferred_element_type=jnp.float32)
    xr_ref[...] = jnp.dot(h1, wr_ref[...], preferred_element_type=jnp.float32)


def _mid(v, b, wl, wr):
    return pl.pallas_call(
        _mid_body,
        grid=(N // NBLK, H),
        in_specs=[
            pl.BlockSpec((H, NBLK, C), lambda i, h: (0, i, 0)),
            pl.BlockSpec((1, C), lambda i, h: (0, 0)),
            pl.BlockSpec((C, C), lambda i, h: (0, h)),
            pl.BlockSpec((C, C), lambda i, h: (0, h)),
        ],
        out_specs=[
            pl.BlockSpec((NBLK, C), lambda i, h: (h * (N // NBLK) + i, 0)),
            pl.BlockSpec((NBLK, C), lambda i, h: (h * (N // NBLK) + i, 0)),
        ],
        out_shape=[
            jax.ShapeDtypeStruct((H * N, C), jnp.float32),
            jax.ShapeDtypeStruct((H * N, C), jnp.float32),
        ],
    )(v, b, wl, wr)


def _post_body(v_ref, b_ref, o_ref):
    o_ref[...] = jnp.mean(v_ref[...], axis=0) + b_ref[...]


def _post(v, b):
    return pl.pallas_call(
        _post_body,
        grid=(N // NBLK,),
        in_specs=[
            pl.BlockSpec((H, NBLK, C), lambda i: (0, i, 0)),
            pl.BlockSpec((1, C), lambda i: (0, 0)),
        ],
        out_specs=pl.BlockSpec((NBLK, C), lambda i: (i, 0)),
        out_shape=jax.ShapeDtypeStruct((N, C), jnp.float32),
    )(v, b)


DROWS_PT = ROWS_PT // 8


def _sc_layer_body(xl_hbm, xr_hbm, src_hbm, dst_hbm, att_hbm, out_hbm,
                   att_v, sdsA, sddA, grxA, sdsB, sddB, grxB,
                   didxSA, didx8A, didxSB, didx8B,
                   xlA, xrA, xlB, xrB, dencA, dencB, denb, finb, zerob,
                   acc, den_acc, semIA, semIB, semLA, semRA, semLB, semRB,
                   semSA, semDA, semSB, semDB):
    c = lax.axis_index("c")
    s = lax.axis_index("s")
    edge_base = s * EPT
    node_base = s * ROWS_PT

    pltpu.sync_copy(att_hbm.at[pl.ds(c * (HEADS_PER_SC * C), HEADS_PER_SC * C)],
                    att_v)

    zero16 = jnp.zeros((16,), jnp.float32)
    lane = lax.iota(jnp.int32, 16)

    def _bcast(v, i):
        return jnp.take_along_axis(v, i, axis=0)

    def _hsum(v):
        for stp in (8, 4, 2, 1):
            v = v + jnp.take_along_axis(v, lane ^ stp, axis=0)
        return v

    def zb_row(r, _):
        for b in range(C // 16):
            zerob[r, pl.ds(b * 16, 16)] = zero16
        return 0
    lax.fori_loop(0, ZB, zb_row, 0)

    def zc_row(r, _):
        for b in range(C // 16):
            dencA[r, pl.ds(b * 16, 16)] = zero16
            dencB[r, pl.ds(b * 16, 16)] = zero16
        return 0
    lax.fori_loop(0, K, zc_row, 0)

    def zero_rows(ref, base, nrows):
        for j in range(nrows // ZB):
            pltpu.sync_copy(zerob, ref.at[pl.ds(base + j * ZB, ZB), :])

    zero_rows(acc, node_base, ROWS_PT)
    zero_rows(den_acc, s * DROWS_PT, DROWS_PT)
    plsc.subcore_barrier()

    def head_body(hl, _):
        h = c * HEADS_PER_SC + hl
        hoff = h * N
        ooff = h * N_PAD
        attv = [att_v[pl.ds(hl * C + b * 16, 16)] for b in range(C // 16)]

        def idx_issue(sds, sdd, semI, base):
            pltpu.async_copy(src_hbm.at[pl.ds(base, K)], sds, semI)
            pltpu.async_copy(dst_hbm.at[pl.ds(base, K)], sdd, semI)

        def idx_wait(sds, sdd, semI):
            pltpu.make_async_copy(src_hbm.at[pl.ds(0, K)], sds, semI).wait()
            pltpu.make_async_copy(dst_hbm.at[pl.ds(0, K)], sdd, semI).wait()

        def addoff(sds, sdd, grx):
            for g in range(K // 16):
                sl = pl.ds(g * 16, 16)
                sds[sl] = sds[sl] + hoff
                grx[sl] = sdd[sl] + hoff

        def gather_issue(sds, grx, xlb, xrb, semL, semR):
            pltpu.async_copy(xl_hbm.at[sds], xlb, semL)
            pltpu.async_copy(xr_hbm.at[grx], xrb, semR)

        def gather_wait(xlb, xrb, semL, semR):
            pltpu.make_async_copy(xl_hbm.at[pl.ds(0, K)], xlb, semL).wait()
            pltpu.make_async_copy(xr_hbm.at[pl.ds(0, K)], xrb, semR).wait()

        def scatter_wait(xlb, denc, semS, semD):
            pltpu.make_async_copy(xlb, acc.at[pl.ds(0, K)], semS).wait()
            pltpu.make_async_copy(denc, den_acc.at[pl.ds(0, K)], semD).wait()

        def compute_scatter(xlb, xrb, sdd, didxS, didx8, denc, semS, semD,
                            base, j):
            @pl.when(j > 0)
            def _():
                scatter_wait(xlb, denc, semS, semD)

            def group_body(g, _):
                d16 = sdd[pl.ds(g * 16, 16)]
                lanesel = d16 & 7
                didxS[pl.ds(g * 16, 16)] = d16
                didx8[pl.ds(g * 16, 16)] = d16 >> 3
                sv = zero16
                for kk in range(16):
                    k = g * 16 + kk
                    accv = zero16
                    for b in range(C // 16):
                        z = xlb[k, pl.ds(b * 16, 16)] + xrb[k, pl.ds(b * 16, 16)]
                        accv = accv + jnp.maximum(z, 0.2 * z) * attv[b]
                    sv = jnp.where(lane == kk, _hsum(accv), sv)
                ids = lane + (base + g * 16)
                ex = jnp.where(ids < E_TOT, jnp.exp(sv), 0.0)
                for kk in range(16):
                    k = g * 16 + kk
                    kkv = jnp.full((16,), kk, jnp.int32)
                    ev = _bcast(ex, kkv)
                    denc[k, pl.ds(0, 16)] = \
                        jnp.where(lane == _bcast(lanesel, kkv), ev, 0.0)
                    for b in range(C // 16):
                        xlb[k, pl.ds(b * 16, 16)] = \
                            ev * xlb[k, pl.ds(b * 16, 16)]
                return 0
            lax.fori_loop(0, K // 16, group_body, 0)
            pltpu.async_copy(xlb, acc.at[didxS], semS, add=True)
            pltpu.async_copy(denc, den_acc.at[didx8], semD, add=True)

        pltpu.sync_copy(src_hbm.at[pl.ds(edge_base, K)], sdsA)
        pltpu.sync_copy(dst_hbm.at[pl.ds(edge_base, K)], sddA)
        addoff(sdsA, sddA, grxA)
        gather_issue(sdsA, grxA, xlA, xrA, semLA, semRA)
        idx_issue(sdsB, sddB, semIB, edge_base + K)

        def pair_body(j, _):
            baseA = edge_base + (2 * j) * K
            more = j < PAIRS - 1
            idx_wait(sdsB, sddB, semIB)
            addoff(sdsB, sddB, grxB)
            gather_issue(sdsB, grxB, xlB, xrB, semLB, semRB)
            gather_wait(xlA, xrA, semLA, semRA)
            compute_scatter(xlA, xrA, sddA, didxSA, didx8A, dencA,
                            semSA, semDA, baseA, j)

            @pl.when(more)
            def _():
                idx_issue(sdsA, sddA, semIA, baseA + 2 * K)

            @pl.when(more)
            def _():
                idx_wait(sdsA, sddA, semIA)
                addoff(sdsA, sddA, grxA)
                gather_issue(sdsA, grxA, xlA, xrA, semLA, semRA)
            gather_wait(xlB, xrB, semLB, semRB)
            compute_scatter(xlB, xrB, sddB, didxSB, didx8B, dencB,
                            semSB, semDB, baseA + K, j)

            @pl.when(more)
            def _():
                idx_issue(sdsB, sddB, semIB, baseA + 3 * K)
            return 0
        lax.fori_loop(0, PAIRS, pair_body, 0)
        scatter_wait(xlA, dencA, semSA, semDA)
        scatter_wait(xlB, dencB, semSB, semDB)
        plsc.subcore_barrier()

        def fin_blk(j, _):
            rb = node_base + j * FIN_B
            pltpu.sync_copy(acc.at[pl.ds(rb, FIN_B), :], finb)
            pltpu.sync_copy(den_acc.at[pl.ds(rb // 8, FIN_B // 8), :], denb)

            def fin_g(g, _):
                ra = denb[2 * g, pl.ds(0, 16)]
                rbv = denb[2 * g + 1, pl.ds(0, 16)]
                shifted = jnp.take_along_axis(rbv, (lane + 8) & 15, axis=0)
                iv16 = 1.0 / jnp.where(lane < 8, ra, shifted)
                for kk in range(16):
                    r = g * 16 + kk
                    iv = _bcast(iv16, jnp.full((16,), kk, jnp.int32))
                    for b in range(C // 16):
                        finb[r, pl.ds(b * 16, 16)] = \
                            finb[r, pl.ds(b * 16, 16)] * iv
                return 0
            lax.fori_loop(0, FIN_B // 16, fin_g, 0)
            pltpu.sync_copy(finb, out_hbm.at[pl.ds(ooff + rb, FIN_B), :])
            zero_rows(acc, rb, FIN_B)
            return 0
        lax.fori_loop(0, ROWS_PT // FIN_B, fin_blk, 0)
        zero_rows(den_acc, s * DROWS_PT, DROWS_PT)
        plsc.subcore_barrier()
        return 0
    lax.fori_loop(0, HEADS_PER_SC, head_body, 0)


def _sc_layer(xl, xr, src, dst, att):
    mesh = plsc.VectorSubcoreMesh(core_axis_name="c", subcore_axis_name="s")
    return pl.kernel(
        _sc_layer_body,
        out_type=jax.ShapeDtypeStruct((H * N_PAD, C), jnp.float32),
        mesh=mesh,
        scratch_types=[
            pltpu.VMEM((HEADS_PER_SC * C,), jnp.float32),
            pltpu.VMEM((K,), jnp.int32),
            pltpu.VMEM((K,), jnp.int32),
            pltpu.VMEM((K,), jnp.int32),
            pltpu.VMEM((K,), jnp.int32),
            pltpu.VMEM((K,), jnp.int32),
            pltpu.VMEM((K,), jnp.int32),
            pltpu.VMEM((K,), jnp.int32),
            pltpu.VMEM((K,), jnp.int32),
            pltpu.VMEM((K,), jnp.int32),
            pltpu.VMEM((K,), jnp.int32),
            pltpu.VMEM((K, C), jnp.float32),
            pltpu.VMEM((K, C), jnp.float32),
            pltpu.VMEM((K, C), jnp.float32),
            pltpu.VMEM((K, C), jnp.float32),
            pltpu.VMEM((K, C), jnp.float32),
            pltpu.VMEM((K, C), jnp.float32),
            pltpu.VMEM((FIN_B // 8, C), jnp.float32),
            pltpu.VMEM((FIN_B, C), jnp.float32),
            pltpu.VMEM((ZB, C), jnp.float32),
            pltpu.VMEM_SHARED((N_PAD, C), jnp.float32),
            pltpu.VMEM_SHARED((N_PAD // 8, C), jnp.float32),
        ] + [pltpu.SemaphoreType.DMA] * 10,
    )(xl, xr, src, dst, att)


def kernel(x, edge_index, W_l1, W_r1, att1, b1, W_l2, W_r2, att2, b2):
    loop = jnp.arange(N, dtype=jnp.int32)
    src = jnp.concatenate([edge_index[0].astype(jnp.int32), loop])
    dst = jnp.concatenate([edge_index[1].astype(jnp.int32), loop])
    src = jnp.pad(src, (0, E_PAD - E_TOT))
    dst = jnp.pad(dst, (0, E_PAD - E_TOT))

    xl1, xr1 = _pre(x, W_l1, W_r1)
    v1 = _sc_layer(xl1, xr1, src, dst, att1.reshape(-1))
    v1 = v1.reshape(H, N_PAD, C)[:, :N, :]
    xl2, xr2 = _mid(v1, b1.reshape(1, C), W_l2, W_r2)
    v2 = _sc_layer(xl2, xr2, src, dst, att2.reshape(-1))
    v2 = v2.reshape(H, N_PAD, C)[:, :N, :]
    return _post(v2, b2.reshape(1, C))
